# baseline (speedup 1.0000x reference)
import functools

import jax
import jax.numpy as jnp
from jax import lax
from jax.experimental import pallas as pl
from jax.experimental.pallas import tpu as pltpu


def _se_pipeline_kernel(x_hbm, w1_ref, w2_ref, o_hbm,
                        xbuf, obuf, in_sem, out_sem, *, n_steps, mb, inv_hw):
    base = pl.program_id(0) * n_steps * mb

    def dma_in(slot, step):
        pltpu.make_async_copy(x_hbm.at[pl.ds(base + step * mb, mb)],
                              xbuf.at[slot], in_sem.at[slot]).start()

    def wait_in(slot):
        pltpu.make_async_copy(xbuf.at[slot], xbuf.at[slot],
                              in_sem.at[slot]).wait()

    def dma_out(slot, step):
        pltpu.make_async_copy(obuf.at[slot], o_hbm.at[pl.ds(base + step * mb, mb)],
                              out_sem.at[slot]).start()

    def wait_out(slot):
        pltpu.make_async_copy(obuf.at[slot], obuf.at[slot],
                              out_sem.at[slot]).wait()

    dma_in(0, 0)

    def body(step, _):
        cur = lax.rem(step, 4)
        nxt = lax.rem(step + 1, 4)

        @pl.when(step + 1 < n_steps)
        def _():
            dma_in(nxt, step + 1)

        wait_in(cur)

        @pl.when(step >= 4)
        def _():
            wait_out(cur)

        for b in range(mb):
            xb = xbuf[cur, b]
            pooled = jnp.sum(xb, axis=1, keepdims=True) * inv_hw
            h = jnp.maximum(
                lax.dot_general(w1_ref[...], pooled,
                                (((1,), (0,)), ((), ())),
                                preferred_element_type=jnp.float32), 0.0)
            s = jax.nn.sigmoid(
                lax.dot_general(w2_ref[...], h, (((1,), (0,)), ((), ())),
                                preferred_element_type=jnp.float32))
            obuf[cur, b] = xb * s
        dma_out(cur, step)
        return ()

    lax.fori_loop(0, n_steps, body, ())
    for tail in range(min(4, n_steps), 0, -1):
        wait_out(lax.rem(n_steps - tail, 4))


def kernel(x, w1, w2):
    B, C, H, W = x.shape
    HW = H * W
    Cr = w1.shape[0]

    x3 = x.reshape(B, C, HW)
    n_cores = 1
    mb = 4 if (B // n_cores) % 4 == 0 else 1
    n_steps = B // (n_cores * mb)

    body = functools.partial(_se_pipeline_kernel, n_steps=n_steps, mb=mb,
                             inv_hw=1.0 / float(HW))
    out3 = pl.pallas_call(
        body,
        out_shape=jax.ShapeDtypeStruct((B, C, HW), x.dtype),
        grid=(n_cores,),
        in_specs=[
            pl.BlockSpec(memory_space=pl.ANY),
            pl.BlockSpec((Cr, C), lambda i: (0, 0)),
            pl.BlockSpec((C, Cr), lambda i: (0, 0)),
        ],
        out_specs=pl.BlockSpec(memory_space=pl.ANY),
        scratch_shapes=[
            pltpu.VMEM((4, mb, C, HW), x.dtype),
            pltpu.VMEM((4, mb, C, HW), x.dtype),
            pltpu.SemaphoreType.DMA((4,)),
            pltpu.SemaphoreType.DMA((4,)),
        ],
        compiler_params=pltpu.CompilerParams(
            dimension_semantics=("parallel",),
            vmem_limit_bytes=64 * 1024 * 1024,
        ),
    )(x3, w1, w2)
    return out3.reshape(B, C, H, W)
